# speedup vs baseline: 1.9831x; 1.9831x over previous
import functools

import jax
import jax.numpy as jnp
from jax import lax
from jax.experimental import pallas as pl
from jax.experimental.pallas import tpu as pltpu
from jax.experimental.pallas import tpu_sc as plsc


def _trig_body(phase_ref, cos_ref, sin_ref):
    p = phase_ref[...]
    cos_ref[...] = jnp.cos(p)
    sin_ref[...] = jnp.sin(p)


def _rel_trig(rel_phase):
    r, d = rel_phase.shape
    return pl.pallas_call(
        _trig_body,
        out_shape=(
            jax.ShapeDtypeStruct((r, d), jnp.float32),
            jax.ShapeDtypeStruct((r, d), jnp.float32),
        ),
    )(rel_phase)


_LANES = 16


def _make_sc_kernel(batch, dim, chunk):
    info = plsc.get_sparse_core_info()
    nc, ns = info.num_cores, info.num_subcores
    nw = nc * ns
    assert batch % (nw * chunk) == 0
    bpw = batch // nw
    n_chunks = bpw // chunk
    mesh = plsc.VectorSubcoreMesh(core_axis_name="c", subcore_axis_name="s")

    f32 = jnp.float32
    out_sds = jax.ShapeDtypeStruct((batch, dim), f32)

    @functools.partial(
        pl.kernel,
        out_type=(out_sds, out_sds, out_sds, out_sds),
        mesh=mesh,
        scratch_types=[
            pltpu.VMEM((chunk,), jnp.int32),
            pltpu.VMEM((chunk,), jnp.int32),
            pltpu.VMEM((chunk,), jnp.int32),
            pltpu.VMEM((chunk, dim), f32),
            pltpu.VMEM((chunk, dim), f32),
            pltpu.VMEM((chunk, dim), f32),
            pltpu.VMEM((chunk, dim), f32),
            pltpu.VMEM((chunk, dim), f32),
            pltpu.VMEM((chunk, dim), f32),
            pltpu.SemaphoreType.DMA,
        ],
    )
    def sc_kernel(h_idx, r_idx, t_idx, ent_re, ent_im, cos_t, sin_t,
                  hr_re_o, hr_im_o, t_re_o, t_im_o,
                  hidx_v, ridx_v, tidx_v, hre_v, him_v, cos_v, sin_v,
                  tre_v, tim_v, sem):
        wid = lax.axis_index("s") * nc + lax.axis_index("c")
        base = wid * bpw

        for cki in range(n_chunks):
            cbase = base + cki * chunk
            sl = pl.ds(cbase, chunk)
            pltpu.sync_copy(h_idx.at[sl], hidx_v)
            pltpu.sync_copy(r_idx.at[sl], ridx_v)
            pltpu.sync_copy(t_idx.at[sl], tidx_v)

            cps = [
                pltpu.async_copy(ent_re.at[hidx_v], hre_v, sem),
                pltpu.async_copy(ent_im.at[hidx_v], him_v, sem),
                pltpu.async_copy(cos_t.at[ridx_v], cos_v, sem),
                pltpu.async_copy(sin_t.at[ridx_v], sin_v, sem),
                pltpu.async_copy(ent_re.at[tidx_v], tre_v, sem),
                pltpu.async_copy(ent_im.at[tidx_v], tim_v, sem),
            ]
            for cp in cps:
                cp.wait()

            pltpu.sync_copy(tre_v, t_re_o.at[sl])
            pltpu.sync_copy(tim_v, t_im_o.at[sl])

            def row_body(r, carry):
                for j in range(dim // _LANES):
                    cs = pl.ds(j * _LANES, _LANES)
                    a = hre_v[r, cs]
                    b = him_v[r, cs]
                    c = cos_v[r, cs]
                    s = sin_v[r, cs]
                    cos_v[r, cs] = a * c - b * s
                    sin_v[r, cs] = a * s + b * c
                return carry

            lax.fori_loop(0, chunk, row_body, 0)

            pltpu.sync_copy(cos_v, hr_re_o.at[sl])
            pltpu.sync_copy(sin_v, hr_im_o.at[sl])

    return sc_kernel


@jax.jit
def kernel(h_idx, r_idx, t_idx, ent_re, ent_im, rel_phase):
    cos_t, sin_t = _rel_trig(rel_phase)
    batch = h_idx.shape[0]
    dim = ent_re.shape[1]
    sc = _make_sc_kernel(batch, dim, chunk=128)
    return sc(h_idx.astype(jnp.int32), r_idx.astype(jnp.int32),
              t_idx.astype(jnp.int32), ent_re, ent_im, cos_t, sin_t)


# speedup vs baseline: 2.2427x; 1.1309x over previous
import functools

import jax
import jax.numpy as jnp
from jax import lax
from jax.experimental import pallas as pl
from jax.experimental.pallas import tpu as pltpu
from jax.experimental.pallas import tpu_sc as plsc


def _trig_body(phase_ref, cos_ref, sin_ref):
    p = phase_ref[...]
    cos_ref[...] = jnp.cos(p)
    sin_ref[...] = jnp.sin(p)


def _rel_trig(rel_phase):
    r, d = rel_phase.shape
    return pl.pallas_call(
        _trig_body,
        out_shape=(
            jax.ShapeDtypeStruct((r, d), jnp.float32),
            jax.ShapeDtypeStruct((r, d), jnp.float32),
        ),
    )(rel_phase)


_LANES = 16


def _make_sc_kernel(batch, dim, chunk):
    info = plsc.get_sparse_core_info()
    nc, ns = info.num_cores, info.num_subcores
    nw = nc * ns
    assert batch % (nw * chunk) == 0
    bpw = batch // nw
    n_chunks = bpw // chunk
    mesh = plsc.VectorSubcoreMesh(core_axis_name="c", subcore_axis_name="s")

    f32 = jnp.float32
    out_sds = jax.ShapeDtypeStruct((batch, dim), f32)
    rows = lambda: pltpu.VMEM((chunk, dim), f32)

    @functools.partial(
        pl.kernel,
        out_type=(out_sds, out_sds, out_sds, out_sds),
        mesh=mesh,
        scratch_types=[
            pltpu.VMEM((n_chunks, chunk), jnp.int32),
            pltpu.VMEM((n_chunks, chunk), jnp.int32),
            pltpu.VMEM((n_chunks, chunk), jnp.int32),
            [rows() for _ in range(2)],
            [rows() for _ in range(2)],
            [rows() for _ in range(2)],
            [rows() for _ in range(2)],
            [rows() for _ in range(2)],
            [rows() for _ in range(2)],
            [pltpu.SemaphoreType.DMA for _ in range(2)],
            [pltpu.SemaphoreType.DMA for _ in range(2)],
        ],
    )
    def sc_kernel(h_idx, r_idx, t_idx, ent_re, ent_im, cos_t, sin_t,
                  hr_re_o, hr_im_o, t_re_o, t_im_o,
                  hidx_v, ridx_v, tidx_v, hre_v, him_v, cos_v, sin_v,
                  tre_v, tim_v, gsem, wsem):
        wid = lax.axis_index("s") * nc + lax.axis_index("c")
        base = wid * bpw
        pltpu.sync_copy(h_idx.at[wid], hidx_v)
        pltpu.sync_copy(r_idx.at[wid], ridx_v)
        pltpu.sync_copy(t_idx.at[wid], tidx_v)

        gd, wd = {}, {}

        def issue_gathers(cki):
            s = cki % 2
            hi, ri, ti = hidx_v.at[cki], ridx_v.at[cki], tidx_v.at[cki]
            gd[s] = [
                pltpu.async_copy(ent_re.at[hi], hre_v[s], gsem[s]),
                pltpu.async_copy(ent_im.at[hi], him_v[s], gsem[s]),
                pltpu.async_copy(cos_t.at[ri], cos_v[s], gsem[s]),
                pltpu.async_copy(sin_t.at[ri], sin_v[s], gsem[s]),
                pltpu.async_copy(ent_re.at[ti], tre_v[s], gsem[s]),
                pltpu.async_copy(ent_im.at[ti], tim_v[s], gsem[s]),
            ]

        issue_gathers(0)
        for cki in range(n_chunks):
            s = cki % 2
            o = (cki + 1) % 2
            if cki + 1 < n_chunks:
                if o in wd:
                    for d in wd.pop(o):
                        d.wait()
                issue_gathers(cki + 1)
            for d in gd.pop(s):
                d.wait()

            hre, him, cos, sin = hre_v[s], him_v[s], cos_v[s], sin_v[s]

            def row_body(r, carry):
                for j in range(dim // _LANES):
                    cs = pl.ds(j * _LANES, _LANES)
                    a = hre[r, cs]
                    b = him[r, cs]
                    c = cos[r, cs]
                    si = sin[r, cs]
                    cos[r, cs] = a * c - b * si
                    sin[r, cs] = a * si + b * c
                return carry

            lax.fori_loop(0, chunk, row_body, 0)

            sl = pl.ds(base + cki * chunk, chunk)
            wd[s] = [
                pltpu.async_copy(cos_v[s], hr_re_o.at[sl], wsem[s]),
                pltpu.async_copy(sin_v[s], hr_im_o.at[sl], wsem[s]),
                pltpu.async_copy(tre_v[s], t_re_o.at[sl], wsem[s]),
                pltpu.async_copy(tim_v[s], t_im_o.at[sl], wsem[s]),
            ]

        for s in (0, 1):
            if s in wd:
                for d in wd.pop(s):
                    d.wait()

    return sc_kernel


@jax.jit
def kernel(h_idx, r_idx, t_idx, ent_re, ent_im, rel_phase):
    cos_t, sin_t = _rel_trig(rel_phase)
    batch = h_idx.shape[0]
    dim = ent_re.shape[1]
    chunk = 64
    info = plsc.get_sparse_core_info()
    nw = info.num_cores * info.num_subcores
    n_chunks = batch // (nw * chunk)
    sc = _make_sc_kernel(batch, dim, chunk)
    shape3 = (nw, n_chunks, chunk)
    return sc(h_idx.astype(jnp.int32).reshape(shape3),
              r_idx.astype(jnp.int32).reshape(shape3),
              t_idx.astype(jnp.int32).reshape(shape3),
              ent_re, ent_im, cos_t, sin_t)


# speedup vs baseline: 2.4337x; 1.0852x over previous
import functools

import jax
import jax.numpy as jnp
from jax import lax
from jax.experimental import pallas as pl
from jax.experimental.pallas import tpu as pltpu
from jax.experimental.pallas import tpu_sc as plsc


def _trig_body(phase_ref, cos_ref, sin_ref):
    p = phase_ref[...]
    cos_ref[...] = jnp.cos(p)
    sin_ref[...] = jnp.sin(p)


def _rel_trig(rel_phase):
    r, d = rel_phase.shape
    return pl.pallas_call(
        _trig_body,
        out_shape=(
            jax.ShapeDtypeStruct((r, d), jnp.float32),
            jax.ShapeDtypeStruct((r, d), jnp.float32),
        ),
    )(rel_phase)


_LANES = 16


def _make_sc_kernel(batch, dim, chunk):
    info = plsc.get_sparse_core_info()
    nc, ns = info.num_cores, info.num_subcores
    nw = nc * ns
    assert batch % (nw * chunk) == 0
    bpw = batch // nw
    n_chunks = bpw // chunk
    mesh = plsc.VectorSubcoreMesh(core_axis_name="c", subcore_axis_name="s")

    f32 = jnp.float32
    out_sds = jax.ShapeDtypeStruct((batch, dim), f32)
    rows = lambda: pltpu.VMEM((chunk, dim), f32)

    @functools.partial(
        pl.kernel,
        out_type=(out_sds, out_sds, out_sds, out_sds),
        mesh=mesh,
        scratch_types=[
            pltpu.VMEM((bpw,), jnp.int32),
            pltpu.VMEM((bpw,), jnp.int32),
            pltpu.VMEM((bpw,), jnp.int32),
            [rows() for _ in range(2)],
            [rows() for _ in range(2)],
            [rows() for _ in range(2)],
            [rows() for _ in range(2)],
            [rows() for _ in range(2)],
            [rows() for _ in range(2)],
            [pltpu.SemaphoreType.DMA for _ in range(2)],
            [pltpu.SemaphoreType.DMA for _ in range(2)],
            pltpu.SemaphoreType.DMA,
        ],
    )
    def sc_kernel(h_idx, r_idx, t_idx, ent_re, ent_im, cos_t, sin_t,
                  hr_re_o, hr_im_o, t_re_o, t_im_o,
                  hidx_v, ridx_v, tidx_v, hre_v, him_v, cos_v, sin_v,
                  tre_v, tim_v, gsem, wsem, isem):
        wid = lax.axis_index("s") * nc + lax.axis_index("c")
        base = wid * bpw
        wsl = pl.ds(base, bpw)
        idx_cps = [
            pltpu.async_copy(h_idx.at[wsl], hidx_v, isem),
            pltpu.async_copy(r_idx.at[wsl], ridx_v, isem),
            pltpu.async_copy(t_idx.at[wsl], tidx_v, isem),
        ]
        for d in idx_cps:
            d.wait()

        gd, wd = {}, {}

        def issue_gathers(cki):
            s = cki % 2
            csl = pl.ds(cki * chunk, chunk)
            hi, ri, ti = hidx_v.at[csl], ridx_v.at[csl], tidx_v.at[csl]
            gd[s] = [
                pltpu.async_copy(ent_re.at[hi], hre_v[s], gsem[s]),
                pltpu.async_copy(ent_im.at[hi], him_v[s], gsem[s]),
                pltpu.async_copy(cos_t.at[ri], cos_v[s], gsem[s]),
                pltpu.async_copy(sin_t.at[ri], sin_v[s], gsem[s]),
                pltpu.async_copy(ent_re.at[ti], tre_v[s], gsem[s]),
                pltpu.async_copy(ent_im.at[ti], tim_v[s], gsem[s]),
            ]

        issue_gathers(0)
        for cki in range(n_chunks):
            s = cki % 2
            o = (cki + 1) % 2
            if cki + 1 < n_chunks:
                if o in wd:
                    for d in wd.pop(o):
                        d.wait()
                issue_gathers(cki + 1)
            for d in gd.pop(s):
                d.wait()

            hre, him, cos, sin = hre_v[s], him_v[s], cos_v[s], sin_v[s]

            def row_body(r, carry):
                for j in range(dim // _LANES):
                    cs = pl.ds(j * _LANES, _LANES)
                    a = hre[r, cs]
                    b = him[r, cs]
                    c = cos[r, cs]
                    si = sin[r, cs]
                    cos[r, cs] = a * c - b * si
                    sin[r, cs] = a * si + b * c
                return carry

            lax.fori_loop(0, chunk, row_body, 0)

            sl = pl.ds(base + cki * chunk, chunk)
            wd[s] = [
                pltpu.async_copy(cos_v[s], hr_re_o.at[sl], wsem[s]),
                pltpu.async_copy(sin_v[s], hr_im_o.at[sl], wsem[s]),
                pltpu.async_copy(tre_v[s], t_re_o.at[sl], wsem[s]),
                pltpu.async_copy(tim_v[s], t_im_o.at[sl], wsem[s]),
            ]

        for s in (0, 1):
            if s in wd:
                for d in wd.pop(s):
                    d.wait()

    return sc_kernel


@jax.jit
def kernel(h_idx, r_idx, t_idx, ent_re, ent_im, rel_phase):
    cos_t, sin_t = _rel_trig(rel_phase)
    batch = h_idx.shape[0]
    dim = ent_re.shape[1]
    sc = _make_sc_kernel(batch, dim, chunk=64)
    return sc(h_idx.astype(jnp.int32), r_idx.astype(jnp.int32),
              t_idx.astype(jnp.int32), ent_re, ent_im, cos_t, sin_t)


# speedup vs baseline: 2.5697x; 1.0559x over previous
import functools

import jax
import jax.numpy as jnp
from jax import lax
from jax.experimental import pallas as pl
from jax.experimental.pallas import tpu as pltpu
from jax.experimental.pallas import tpu_sc as plsc


_FIX = 32767.0


def _trig_body(phase_ref, packed_ref):
    p = phase_ref[...]
    c = jnp.round(jnp.cos(p) * _FIX).astype(jnp.int32)
    s = jnp.round(jnp.sin(p) * _FIX).astype(jnp.int32)
    packed_ref[...] = (c & 0xFFFF) | (s << 16)


def _rel_trig_packed(rel_phase):
    r, d = rel_phase.shape
    return pl.pallas_call(
        _trig_body,
        out_shape=jax.ShapeDtypeStruct((r, d), jnp.int32),
    )(rel_phase)


_LANES = 16


def _make_sc_kernel(batch, dim, chunk, nbuf, dist):
    info = plsc.get_sparse_core_info()
    nc, ns = info.num_cores, info.num_subcores
    nw = nc * ns
    assert batch % (nw * chunk) == 0
    assert dist < nbuf
    bpw = batch // nw
    n_chunks = bpw // chunk
    mesh = plsc.VectorSubcoreMesh(core_axis_name="c", subcore_axis_name="s")

    f32 = jnp.float32
    out_sds = jax.ShapeDtypeStruct((batch, dim), f32)
    rows = lambda dt: pltpu.VMEM((chunk, dim), dt)
    inv_fix = jnp.float32(1.0 / _FIX)

    @functools.partial(
        pl.kernel,
        out_type=(out_sds, out_sds, out_sds, out_sds),
        mesh=mesh,
        scratch_types=[
            pltpu.VMEM((bpw,), jnp.int32),
            pltpu.VMEM((bpw,), jnp.int32),
            pltpu.VMEM((bpw,), jnp.int32),
            [rows(f32) for _ in range(nbuf)],
            [rows(f32) for _ in range(nbuf)],
            [rows(jnp.int32) for _ in range(nbuf)],
            [rows(f32) for _ in range(nbuf)],
            [rows(f32) for _ in range(nbuf)],
            [pltpu.SemaphoreType.DMA for _ in range(nbuf)],
            [pltpu.SemaphoreType.DMA for _ in range(nbuf)],
            [pltpu.SemaphoreType.DMA for _ in range(nbuf)],
            pltpu.SemaphoreType.DMA,
        ],
    )
    def sc_kernel(h_idx, r_idx, t_idx, ent_re, ent_im, trig_t,
                  hr_re_o, hr_im_o, t_re_o, t_im_o,
                  hidx_v, ridx_v, tidx_v, hre_v, him_v, pk_v,
                  tre_v, tim_v, gsem, tsem, wsem, isem):
        wid = lax.axis_index("s") * nc + lax.axis_index("c")
        base = wid * bpw
        wsl = pl.ds(base, bpw)
        idx_cps = [
            pltpu.async_copy(h_idx.at[wsl], hidx_v, isem),
            pltpu.async_copy(r_idx.at[wsl], ridx_v, isem),
            pltpu.async_copy(t_idx.at[wsl], tidx_v, isem),
        ]
        for d in idx_cps:
            d.wait()

        gd, td, wd = {}, {}, {}

        def issue_gathers(cki):
            s = cki % nbuf
            csl = pl.ds(cki * chunk, chunk)
            hi, ri, ti = hidx_v.at[csl], ridx_v.at[csl], tidx_v.at[csl]
            gd[s] = [
                pltpu.async_copy(ent_re.at[hi], hre_v[s], gsem[s]),
                pltpu.async_copy(ent_im.at[hi], him_v[s], gsem[s]),
                pltpu.async_copy(trig_t.at[ri], pk_v[s], gsem[s]),
            ]
            td[s] = [
                pltpu.async_copy(ent_re.at[ti], tre_v[s], tsem[s]),
                pltpu.async_copy(ent_im.at[ti], tim_v[s], tsem[s]),
            ]

        for g in range(min(dist, n_chunks)):
            issue_gathers(g)
        for cki in range(n_chunks):
            g = cki + dist
            if g < n_chunks:
                so = g % nbuf
                if so in wd:
                    for d in wd.pop(so):
                        d.wait()
                issue_gathers(g)

            s = cki % nbuf
            sl = pl.ds(base + cki * chunk, chunk)
            for d in td.pop(s):
                d.wait()
            wr = [
                pltpu.async_copy(tre_v[s], t_re_o.at[sl], wsem[s]),
                pltpu.async_copy(tim_v[s], t_im_o.at[sl], wsem[s]),
            ]
            for d in gd.pop(s):
                d.wait()

            hre, him, pk = hre_v[s], him_v[s], pk_v[s]

            def row_body(r, carry):
                for j in range(dim // _LANES):
                    cs = pl.ds(j * _LANES, _LANES)
                    a = hre[r, cs]
                    b = him[r, cs]
                    x = pk[r, cs]
                    c = lax.shift_right_arithmetic(
                        lax.shift_left(x, 16), 16).astype(f32)
                    si = lax.shift_right_arithmetic(x, 16).astype(f32)
                    hre[r, cs] = (a * c - b * si) * inv_fix
                    him[r, cs] = (a * si + b * c) * inv_fix
                return carry

            lax.fori_loop(0, chunk, row_body, 0)

            wd[s] = wr + [
                pltpu.async_copy(hre_v[s], hr_re_o.at[sl], wsem[s]),
                pltpu.async_copy(him_v[s], hr_im_o.at[sl], wsem[s]),
            ]

        for s in list(wd):
            for d in wd.pop(s):
                d.wait()

    return sc_kernel


@jax.jit
def kernel(h_idx, r_idx, t_idx, ent_re, ent_im, rel_phase):
    batch = h_idx.shape[0]
    dim = ent_re.shape[1]
    trig_t = _rel_trig_packed(rel_phase)
    sc = _make_sc_kernel(batch, dim, chunk=32, nbuf=4, dist=2)
    return sc(h_idx.astype(jnp.int32), r_idx.astype(jnp.int32),
              t_idx.astype(jnp.int32), ent_re, ent_im, trig_t)


# speedup vs baseline: 2.6173x; 1.0185x over previous
import functools

import jax
import jax.numpy as jnp
from jax import lax
from jax.experimental import pallas as pl
from jax.experimental.pallas import tpu as pltpu
from jax.experimental.pallas import tpu_sc as plsc


_FIX = 32767.0


def _trig_body(phase_ref, packed_ref):
    p = phase_ref[...]
    c = jnp.round(jnp.cos(p) * _FIX).astype(jnp.int32)
    s = jnp.round(jnp.sin(p) * _FIX).astype(jnp.int32)
    packed_ref[...] = (c & 0xFFFF) | (s << 16)


def _rel_trig_packed(rel_phase):
    r, d = rel_phase.shape
    return pl.pallas_call(
        _trig_body,
        out_shape=jax.ShapeDtypeStruct((r, d), jnp.int32),
    )(rel_phase)


_LANES = 16


def _make_sc_kernel(batch, dim, chunk, nbuf, dist):
    info = plsc.get_sparse_core_info()
    nc, ns = info.num_cores, info.num_subcores
    nw = nc * ns
    assert batch % (nw * chunk) == 0
    assert dist < nbuf
    bpw = batch // nw
    n_chunks = bpw // chunk
    mesh = plsc.VectorSubcoreMesh(core_axis_name="c", subcore_axis_name="s")

    f32 = jnp.float32
    out_sds = jax.ShapeDtypeStruct((batch, dim), f32)
    rows = lambda dt: pltpu.VMEM((chunk, dim), dt)
    inv_fix = jnp.float32(1.0 / _FIX)

    @functools.partial(
        pl.kernel,
        out_type=(out_sds, out_sds, out_sds, out_sds),
        mesh=mesh,
        scratch_types=[
            pltpu.VMEM((bpw,), jnp.int32),
            pltpu.VMEM((bpw,), jnp.int32),
            pltpu.VMEM((bpw,), jnp.int32),
            [rows(f32) for _ in range(nbuf)],
            [rows(f32) for _ in range(nbuf)],
            [rows(jnp.int32) for _ in range(nbuf)],
            [rows(f32) for _ in range(nbuf)],
            [rows(f32) for _ in range(nbuf)],
            [pltpu.SemaphoreType.DMA for _ in range(nbuf)],
            [pltpu.SemaphoreType.DMA for _ in range(nbuf)],
            [pltpu.SemaphoreType.DMA for _ in range(nbuf)],
            pltpu.SemaphoreType.DMA,
        ],
    )
    def sc_kernel(h_idx, r_idx, t_idx, ent_re, ent_im, trig_t,
                  hr_re_o, hr_im_o, t_re_o, t_im_o,
                  hidx_v, ridx_v, tidx_v, hre_v, him_v, pk_v,
                  tre_v, tim_v, gsem, tsem, wsem, isem):
        wid = lax.axis_index("s") * nc + lax.axis_index("c")
        base = wid * bpw
        wsl = pl.ds(base, bpw)
        idx_cps = [
            pltpu.async_copy(h_idx.at[wsl], hidx_v, isem),
            pltpu.async_copy(r_idx.at[wsl], ridx_v, isem),
            pltpu.async_copy(t_idx.at[wsl], tidx_v, isem),
        ]
        for d in idx_cps:
            d.wait()

        gd, td, wd = {}, {}, {}

        def issue_gathers(cki):
            s = cki % nbuf
            csl = pl.ds(cki * chunk, chunk)
            hi, ri, ti = hidx_v.at[csl], ridx_v.at[csl], tidx_v.at[csl]
            gd[s] = [
                pltpu.async_copy(ent_re.at[hi], hre_v[s], gsem[s]),
                pltpu.async_copy(ent_im.at[hi], him_v[s], gsem[s]),
                pltpu.async_copy(trig_t.at[ri], pk_v[s], gsem[s]),
            ]
            td[s] = [
                pltpu.async_copy(ent_re.at[ti], tre_v[s], tsem[s]),
                pltpu.async_copy(ent_im.at[ti], tim_v[s], tsem[s]),
            ]

        for g in range(min(dist, n_chunks)):
            issue_gathers(g)
        for cki in range(n_chunks):
            g = cki + dist
            if g < n_chunks:
                so = g % nbuf
                if so in wd:
                    for d in wd.pop(so):
                        d.wait()
                issue_gathers(g)

            s = cki % nbuf
            sl = pl.ds(base + cki * chunk, chunk)
            for d in td.pop(s):
                d.wait()
            wr = [
                pltpu.async_copy(tre_v[s], t_re_o.at[sl], wsem[s]),
                pltpu.async_copy(tim_v[s], t_im_o.at[sl], wsem[s]),
            ]
            for d in gd.pop(s):
                d.wait()

            hre, him, pk = hre_v[s], him_v[s], pk_v[s]

            def row_body(r, carry):
                for j in range(dim // _LANES):
                    cs = pl.ds(j * _LANES, _LANES)
                    a = hre[r, cs]
                    b = him[r, cs]
                    x = pk[r, cs]
                    c = lax.shift_right_arithmetic(
                        lax.shift_left(x, 16), 16).astype(f32)
                    si = lax.shift_right_arithmetic(x, 16).astype(f32)
                    hre[r, cs] = (a * c - b * si) * inv_fix
                    him[r, cs] = (a * si + b * c) * inv_fix
                return carry

            lax.fori_loop(0, chunk, row_body, 0)

            wd[s] = wr + [
                pltpu.async_copy(hre_v[s], hr_re_o.at[sl], wsem[s]),
                pltpu.async_copy(him_v[s], hr_im_o.at[sl], wsem[s]),
            ]

        for s in list(wd):
            for d in wd.pop(s):
                d.wait()

    return sc_kernel


@jax.jit
def kernel(h_idx, r_idx, t_idx, ent_re, ent_im, rel_phase):
    batch = h_idx.shape[0]
    dim = ent_re.shape[1]
    trig_t = _rel_trig_packed(rel_phase)
    sc = _make_sc_kernel(batch, dim, chunk=64, nbuf=3, dist=2)
    return sc(h_idx.astype(jnp.int32), r_idx.astype(jnp.int32),
              t_idx.astype(jnp.int32), ent_re, ent_im, trig_t)
